# baseline (speedup 1.0000x reference)
import functools
import math

import jax
import jax.numpy as jnp
from jax import lax
from jax.experimental import pallas as pl
from jax.experimental.pallas import tpu as pltpu
from jax.experimental.pallas import tpu_sc as plsc

_D = 1024
_H = _D // 2
_NA = 4
_L = 16
_CS = 16
_NBI = 3
_NBO = 2
_G = 6


def _sc_call(x2, attr_embed, elem_embed, B, S):
    NROWS = B * S
    info = plsc.get_sparse_core_info()
    NC, NS = info.num_cores, info.num_subcores
    NW = NC * NS
    SPW = S // NW
    EPW = SPW // _NA
    CPB = SPW // _CS
    T = B * CPB
    scale = math.sqrt(_D)

    mesh = plsc.VectorSubcoreMesh(core_axis_name="c", subcore_axis_name="s")

    @functools.partial(
        pl.kernel,
        out_type=jax.ShapeDtypeStruct((NROWS, _D), jnp.float32),
        mesh=mesh,
        scratch_types=[
            pltpu.VMEM((_NA, _H), jnp.float32),
            pltpu.VMEM((EPW, _H), jnp.float32),
            pltpu.VMEM((_NBI, _CS, _D), jnp.float32),
            pltpu.VMEM((_NBO, _CS, _D), jnp.float32),
            pltpu.SemaphoreType.DMA,
            pltpu.SemaphoreType.DMA,
            pltpu.SemaphoreType.DMA,
            pltpu.SemaphoreType.DMA,
            pltpu.SemaphoreType.DMA,
        ],
    )
    def body(x_hbm, attr_hbm, elem_hbm, out_hbm, attr_v, elem_v, xb, ob,
             si0, si1, si2, so0, so1):
        wid = lax.axis_index("s") * NC + lax.axis_index("c")
        base = wid * SPW
        in_sems = [si0, si1, si2]
        out_sems = [so0, so1]

        pltpu.sync_copy(attr_hbm, attr_v)
        pltpu.sync_copy(elem_hbm.at[pl.ds(wid * EPW, EPW)], elem_v)

        def hbm_row0(t):
            b = t // CPB
            c = lax.rem(t, CPB)
            return b * S + base + c * _CS

        def in_copy(t, k):
            return pltpu.make_async_copy(
                x_hbm.at[pl.ds(hbm_row0(t), _CS)], xb.at[k], in_sems[k])

        def out_copy(t, k):
            return pltpu.make_async_copy(
                ob.at[k], out_hbm.at[pl.ds(hbm_row0(t), _CS)], out_sems[k])

        def compute(ki, ko, t):
            c = lax.rem(t, CPB)
            er0 = c * (_CS // _NA)

            @plsc.parallel_loop(0, _H // _L, unroll=2)
            def attr_cols(cc):
                col = cc * _L
                a = [attr_v[i, pl.ds(col, _L)] for i in range(_NA)]
                for r in range(_CS):
                    ob[ko, r, pl.ds(col, _L)] = (
                        xb[ki, r, pl.ds(col, _L)] * scale + a[r % _NA])

            @plsc.parallel_loop(0, _H // _L, unroll=2)
            def elem_cols(cc):
                col = cc * _L
                for r4 in range(_CS // _NA):
                    e = elem_v[er0 + r4, pl.ds(col, _L)]
                    for i in range(_NA):
                        r = r4 * _NA + i
                        ob[ko, r, pl.ds(_H + col, _L)] = (
                            xb[ki, r, pl.ds(_H + col, _L)] * scale + e)

        def chunk_step(t, ki, ko, static):
            if static:
                if t >= _NBO:
                    out_copy(t - _NBO, ko).wait()
            else:
                @pl.when(t >= _NBO)
                def _wait_prev_out():
                    out_copy(t - _NBO, ko).wait()

            in_copy(t, ki).wait()
            compute(ki, ko, t)

            if static:
                if t + _NBI < T:
                    in_copy(t + _NBI, ki).start()
            else:
                @pl.when(t + _NBI < T)
                def _prefetch():
                    in_copy(t + _NBI, ki).start()

            out_copy(t, ko).start()

        for k in range(_NBI):
            in_copy(k, k).start()

        def group(g, _):
            tt = g * _G
            for j in range(_G):
                chunk_step(tt + j, j % _NBI, j % _NBO, static=False)
            return 0

        lax.fori_loop(0, T // _G, group, 0)

        for t in range(T - T % _G, T):
            chunk_step(t, t % _NBI, t % _NBO, static=True)

        for k in range(_NBO):
            out_copy(T - _NBO + k, k).wait()

    return body(x2, attr_embed, elem_embed)


def kernel(x, attr_embed, elem_embed):
    B, S, D = x.shape
    x2 = x.reshape(B * S, D)
    out = _sc_call(x2, attr_embed, elem_embed, B, S)
    return out.reshape(B, S, D)


# speedup vs baseline: 1.1793x; 1.1793x over previous
import functools
import math

import jax
import jax.numpy as jnp
from jax import lax
from jax.experimental import pallas as pl
from jax.experimental.pallas import tpu as pltpu
from jax.experimental.pallas import tpu_sc as plsc

_D = 1024
_H = _D // 2
_NA = 4
_L = 16
_CS = 16
_NBI = 3
_NBO = 2


def _sc_call(x2, attr_embed, elem_embed, B, S):
    NROWS = B * S
    info = plsc.get_sparse_core_info()
    NC, NS = info.num_cores, info.num_subcores
    NW = NC * NS
    SPW = S // NW
    EPW = SPW // _NA
    CPB = SPW // _CS
    T = B * CPB
    scale = math.sqrt(_D)

    mesh = plsc.VectorSubcoreMesh(core_axis_name="c", subcore_axis_name="s")

    @functools.partial(
        pl.kernel,
        out_type=jax.ShapeDtypeStruct((NROWS, _D), jnp.float32),
        mesh=mesh,
        scratch_types=[
            pltpu.VMEM((_NA, _H), jnp.float32),
            pltpu.VMEM((EPW, _H), jnp.float32),
            pltpu.VMEM((_NBI, _CS, _D), jnp.float32),
            pltpu.VMEM((_NBO, _CS, _D), jnp.float32),
            pltpu.SemaphoreType.DMA((_NBI,)),
            pltpu.SemaphoreType.DMA((_NBO,)),
        ],
    )
    def body(x_hbm, attr_hbm, elem_hbm, out_hbm, attr_v, elem_v, xb, ob,
             in_sem, out_sem):
        wid = lax.axis_index("s") * NC + lax.axis_index("c")
        base = wid * SPW

        pltpu.sync_copy(attr_hbm, attr_v)
        pltpu.sync_copy(elem_hbm.at[pl.ds(wid * EPW, EPW)], elem_v)

        def hbm_row0(t):
            b = t // CPB
            c = lax.rem(t, CPB)
            return b * S + base + c * _CS

        def in_copy(t, k):
            return pltpu.make_async_copy(
                x_hbm.at[pl.ds(hbm_row0(t), _CS)], xb.at[k], in_sem.at[k])

        def out_copy(t, k):
            return pltpu.make_async_copy(
                ob.at[k], out_hbm.at[pl.ds(hbm_row0(t), _CS)], out_sem.at[k])

        def compute(ki, ko, t):
            c = lax.rem(t, CPB)
            er0 = c * (_CS // _NA)

            @plsc.parallel_loop(0, _H // _L, unroll=2)
            def attr_cols(cc):
                col = cc * _L
                a = [attr_v[i, pl.ds(col, _L)] for i in range(_NA)]
                for r in range(_CS):
                    ob[ko, r, pl.ds(col, _L)] = (
                        xb[ki, r, pl.ds(col, _L)] * scale + a[r % _NA])

            @plsc.parallel_loop(0, _H // _L, unroll=2)
            def elem_cols(cc):
                col = cc * _L
                for r4 in range(_CS // _NA):
                    e = elem_v[er0 + r4, pl.ds(col, _L)]
                    for i in range(_NA):
                        r = r4 * _NA + i
                        ob[ko, r, pl.ds(_H + col, _L)] = (
                            xb[ki, r, pl.ds(_H + col, _L)] * scale + e)

        for k in range(_NBI):
            in_copy(k, k).start()

        def step(t, _):
            ki = lax.rem(t, _NBI)
            ko = lax.rem(t, _NBO)

            @pl.when(t >= _NBO)
            def _wait_prev_out():
                out_copy(t - _NBO, ko).wait()

            in_copy(t, ki).wait()
            compute(ki, ko, t)

            @pl.when(t + _NBI < T)
            def _prefetch():
                in_copy(t + _NBI, ki).start()

            out_copy(t, ko).start()
            return 0

        lax.fori_loop(0, T, step, 0)

        for t in range(T - _NBO, T):
            out_copy(t, lax.rem(t, _NBO)).wait()

    return body(x2, attr_embed, elem_embed)


def kernel(x, attr_embed, elem_embed):
    B, S, D = x.shape
    x2 = x.reshape(B * S, D)
    out = _sc_call(x2, attr_embed, elem_embed, B, S)
    return out.reshape(B, S, D)
